# baseline (speedup 1.0000x reference)
import functools

import jax
import jax.numpy as jnp
from jax import lax
from jax.experimental import pallas as pl
from jax.experimental.pallas import tpu as pltpu
from jax.experimental.pallas import tpu_sc as plsc

_ETA_OPT = 1000000.0

_NC = 2
_NS = 16
_NW = _NC * _NS
_L = 16

_NBUF = 3
_NCUT = 512


def _permute(v, idx):
    return lax.gather(
        v, idx[:, None],
        dimension_numbers=lax.GatherDimensionNumbers(
            offset_dims=(), collapsed_slice_dims=(0,), start_index_map=(0,)),
        slice_sizes=(1,),
        mode=lax.GatherScatterMode.PROMISE_IN_BOUNDS)


def _sc_axb_body(a_hbm, at_hbm, q_hbm, s_hbm, b_hbm, out_hbm,
                 xv, yf, yv, bv, iyv, buf0, buf1, buf2, sv,
                 sem0, sem1, sem2, *, rows_per_w, r_grp, pg_per_w):
    wid = lax.axis_index("s") * _NC + lax.axis_index("c")
    row0 = wid * rows_per_w
    n = xv.shape[0]
    nj = n // _L
    ngrp = rows_per_w // r_grp
    bufs = (buf0, buf1, buf2)
    sems = (sem0, sem1, sem2)

    pltpu.sync_copy(s_hbm.at[pl.ds(0, n)], xv)
    pltpu.sync_copy(s_hbm.at[pl.ds(n, n)], yf)
    pltpu.sync_copy(s_hbm.at[pl.ds(n + row0, rows_per_w)], yv)
    pltpu.sync_copy(b_hbm.at[pl.ds(row0, rows_per_w)], bv)
    pltpu.sync_copy(s_hbm.at[pl.ds(2 * n + row0, rows_per_w)], iyv)

    for p in range(min(_NBUF - 1, ngrp)):
        pltpu.make_async_copy(
            a_hbm.at[pl.ds(row0 + p * r_grp, r_grp), :], bufs[p],
            sems[p]).start()

    lane = lax.iota(jnp.int32, _L)
    svec = jnp.zeros((_L,), jnp.float32)
    cur16 = jnp.zeros((_L,), jnp.float32)
    for g in range(ngrp):
        cur = bufs[g % _NBUF]
        pltpu.make_async_copy(
            a_hbm.at[pl.ds(row0 + g * r_grp, r_grp), :], cur,
            sems[g % _NBUF]).wait()
        nxt = g + _NBUF - 1
        if nxt < ngrp:
            pltpu.make_async_copy(
                a_hbm.at[pl.ds(row0 + nxt * r_grp, r_grp), :],
                bufs[nxt % _NBUF], sems[nxt % _NBUF]).start()

        zero = jnp.zeros((_L,), jnp.float32)
        accs_init = (zero,) * r_grp
        unroll = 4

        def jbody(jj, accs, cur=cur):
            for u in range(unroll):
                col = (jj * unroll + u) * _L
                xc = xv[pl.ds(col, _L)]
                accs = tuple(
                    accs[r] + cur[r, pl.ds(col, _L)] * xc
                    for r in range(r_grp))
            return accs

        accs = lax.fori_loop(0, nj // unroll, jbody, accs_init)
        for r in range(r_grp):
            row = g * r_grp + r
            v = accs[r]
            for k in (8, 4, 2, 1):
                v = v + _permute(v, lane ^ k)
            cur16 = jnp.where(lane == (row % _L), v, cur16)
            if (row + 1) % _L == 0:
                base = row + 1 - _L
                t = yv[pl.ds(base, _L)] * (cur16 - bv[pl.ds(base, _L)])
                t = t * iyv[pl.ds(base, _L)]
                svec = svec + jnp.abs(t)
                cur16 = jnp.zeros((_L,), jnp.float32)

    prow0 = wid * _L
    pltpu.sync_copy(s_hbm.at[pl.ds(3 * n + prow0, _L)], sv)
    cv = sv[...]
    pltpu.sync_copy(s_hbm.at[pl.ds(4 * n + prow0, _L)], sv)
    ilv = sv[...]
    pltpu.sync_copy(s_hbm.at[pl.ds(5 * n + prow0, _L)], sv)
    iuv = sv[...]
    pltpu.sync_copy(s_hbm.at[pl.ds(6 * n + prow0, _L)], sv)
    lv = sv[...]
    pltpu.sync_copy(s_hbm.at[pl.ds(7 * n + prow0, _L)], sv)
    uv = sv[...]

    npg = pg_per_w // 4
    for p in range(min(_NBUF - 1, npg)):
        pltpu.make_async_copy(
            at_hbm.at[pl.ds(prow0 + p * 4, 4), :],
            bufs[p].at[pl.ds(0, 4), :], sems[p]).start()
        pltpu.make_async_copy(
            q_hbm.at[pl.ds(prow0 + p * 4, 4), :],
            bufs[p].at[pl.ds(4, 4), :], sems[p]).start()

    curA = jnp.zeros((_L,), jnp.float32)
    curQ = jnp.zeros((_L,), jnp.float32)
    for h in range(npg):
        cur = bufs[h % _NBUF]
        pltpu.make_async_copy(
            at_hbm.at[pl.ds(prow0 + h * 4, 4), :],
            cur.at[pl.ds(0, 4), :], sems[h % _NBUF]).wait()
        pltpu.make_async_copy(
            q_hbm.at[pl.ds(prow0 + h * 4, 4), :],
            cur.at[pl.ds(4, 4), :], sems[h % _NBUF]).wait()
        nxt = h + _NBUF - 1
        if nxt < npg:
            pltpu.make_async_copy(
                at_hbm.at[pl.ds(prow0 + nxt * 4, 4), :],
                bufs[nxt % _NBUF].at[pl.ds(0, 4), :],
                sems[nxt % _NBUF]).start()
            pltpu.make_async_copy(
                q_hbm.at[pl.ds(prow0 + nxt * 4, 4), :],
                bufs[nxt % _NBUF].at[pl.ds(4, 4), :],
                sems[nxt % _NBUF]).start()

        zero = jnp.zeros((_L,), jnp.float32)
        paccs = (zero,) * 8

        def pbody(jj, accs, cur=cur):
            for u in range(4):
                col = (jj * 4 + u) * _L
                yc = yf[pl.ds(col, _L)]
                xc = xv[pl.ds(col, _L)]
                accs = tuple(
                    accs[r] + cur[r, pl.ds(col, _L)] * (yc if r < 4 else xc)
                    for r in range(8))
            return accs

        paccs = lax.fori_loop(0, nj // 4, pbody, paccs)
        for r in range(4):
            vA = paccs[r]
            vQ = paccs[4 + r]
            for k in (8, 4, 2, 1):
                vA = vA + _permute(vA, lane ^ k)
                vQ = vQ + _permute(vQ, lane ^ k)
            pos = h * 4 + r
            curA = jnp.where(lane == pos, vA, curA)
            curQ = jnp.where(lane == pos, vQ, curQ)

    x16 = xv[pl.ds(prow0, _L)]
    pgv = cv - curA + curQ
    lbv = jnp.maximum(pgv, 0.0) * ilv
    svec = svec + jnp.abs((x16 - lv) * lbv)
    ubv = jnp.maximum(-pgv, 0.0) * iuv
    svec = svec + jnp.abs((uv - x16) * ubv)

    sv[...] = svec
    pltpu.sync_copy(sv, out_hbm.at[wid])


def _sc_axb(A, AT, Q, svec, b):
    m, n = A.shape
    rows_per_w = m // _NW
    pg_per_w = _NCUT // _NW
    r_grp = 8
    mesh = plsc.VectorSubcoreMesh(core_axis_name="c", subcore_axis_name="s")
    f = pl.kernel(
        functools.partial(_sc_axb_body, rows_per_w=rows_per_w, r_grp=r_grp,
                          pg_per_w=pg_per_w),
        mesh=mesh,
        out_type=jax.ShapeDtypeStruct((_NW, _L), jnp.float32),
        scratch_types=[
            pltpu.VMEM((n,), jnp.float32),
            pltpu.VMEM((n,), jnp.float32),
            pltpu.VMEM((rows_per_w,), jnp.float32),
            pltpu.VMEM((rows_per_w,), jnp.float32),
            pltpu.VMEM((rows_per_w,), jnp.float32),
            pltpu.VMEM((r_grp, n), jnp.float32),
            pltpu.VMEM((r_grp, n), jnp.float32),
            pltpu.VMEM((r_grp, n), jnp.float32),
            pltpu.VMEM((_L,), jnp.float32),
            pltpu.SemaphoreType.DMA,
            pltpu.SemaphoreType.DMA,
            pltpu.SemaphoreType.DMA,
        ],
    )
    return f(A, AT, Q, svec, b)


_SPLIT = 2


def _tc_body(at0_ref, at1_ref, q0_ref, q1_ref, w_ref, out_ref, acc_ref,
             *, blk, nsteps):
    i = pl.program_id(0)

    xv = w_ref[:, 0:1]
    yv = w_ref[:, 1:2]

    s = jnp.float32(0.0)
    for k, (at_ref, q_ref) in enumerate(((at0_ref, q0_ref),
                                         (at1_ref, q1_ref))):
        aty = jnp.dot(at_ref[...], yv, preferred_element_type=jnp.float32)
        qx = jnp.dot(q_ref[...], xv, preferred_element_type=jnp.float32)

        wb = w_ref[pl.ds(_NCUT + (_SPLIT * i + k) * blk, blk), :]
        pg = wb[:, 2:3] - aty + qx

        x_blk = wb[:, 0:1]
        lb = jnp.maximum(pg, 0.0) * wb[:, 3:4]
        s = s + jnp.sum(jnp.abs((x_blk - wb[:, 5:6]) * lb))
        ub = jnp.maximum(-pg, 0.0) * wb[:, 4:5]
        s = s + jnp.sum(jnp.abs((wb[:, 6:7] - x_blk) * ub))

    @pl.when(i == 0)
    def _():
        acc_ref[0] = 0.0

    acc_ref[0] += s

    @pl.when(i == nsteps - 1)
    def _():
        out_ref[...] = jnp.full((1, 1), acc_ref[0], dtype=jnp.float32)


def _tc_rc(Q, AT, W):
    n = Q.shape[0]
    m = AT.shape[1]
    blk = 256
    off = _NCUT // blk
    nsteps = (n - _NCUT) // (blk * _SPLIT)

    spec = lambda cols, k: pl.BlockSpec(
        (blk, cols), lambda i, k=k: (off + _SPLIT * i + k, 0))

    return pl.pallas_call(
        functools.partial(_tc_body, blk=blk, nsteps=nsteps),
        grid=(nsteps,),
        in_specs=[
            spec(m, 0),
            spec(m, 1),
            spec(n, 0),
            spec(n, 1),
            pl.BlockSpec((n, 7), lambda i: (0, 0)),
        ],
        out_specs=pl.BlockSpec((1, 1), lambda i: (0, 0)),
        out_shape=jax.ShapeDtypeStruct((1, 1), jnp.float32),
        scratch_shapes=[pltpu.SMEM((1,), jnp.float32)],
    )(AT, AT, Q, Q, W)


def kernel(Q, A, AT, b, c, x, y, Iy, il, iu, l, u):
    svec = jnp.concatenate([x[:, 0], y[:, 0], Iy[:, 0], c,
                            il[:, 0], iu[:, 0], l[:, 0], u[:, 0]])
    W = jnp.concatenate([x, y, c[:, None], il, iu, l, u], axis=1)
    sc_out = _sc_axb(A, AT, Q, svec, b)
    tc_out = _tc_rc(Q, AT, W)
    return (jnp.sum(sc_out) + tc_out[0, 0]) * (1.0 / _ETA_OPT)


# speedup vs baseline: 1.1034x; 1.1034x over previous
import functools

import jax
import jax.numpy as jnp
from jax import lax
from jax.experimental import pallas as pl
from jax.experimental.pallas import tpu as pltpu
from jax.experimental.pallas import tpu_sc as plsc

_ETA_OPT = 1000000.0

_NC = 2
_NS = 16
_NW = _NC * _NS
_L = 16

_NBUF = 3


def _permute(v, idx):
    return lax.gather(
        v, idx[:, None],
        dimension_numbers=lax.GatherDimensionNumbers(
            offset_dims=(), collapsed_slice_dims=(0,), start_index_map=(0,)),
        slice_sizes=(1,),
        mode=lax.GatherScatterMode.PROMISE_IN_BOUNDS)


def _sc_axb_body(a_hbm, s_hbm, b_hbm, out_hbm,
                 xv, yv, bv, iyv, buf0, buf1, buf2, sv,
                 sem0, sem1, sem2, *, rows_per_w, r_grp):
    wid = lax.axis_index("s") * _NC + lax.axis_index("c")
    row0 = wid * rows_per_w
    n = xv.shape[0]
    nj = n // _L
    ngrp = rows_per_w // r_grp
    bufs = (buf0, buf1, buf2)
    sems = (sem0, sem1, sem2)

    pltpu.sync_copy(s_hbm.at[pl.ds(0, n)], xv)
    pltpu.sync_copy(s_hbm.at[pl.ds(n + row0, rows_per_w)], yv)
    pltpu.sync_copy(b_hbm.at[pl.ds(row0, rows_per_w)], bv)
    pltpu.sync_copy(s_hbm.at[pl.ds(2 * n + row0, rows_per_w)], iyv)

    for p in range(min(_NBUF - 1, ngrp)):
        pltpu.make_async_copy(
            a_hbm.at[pl.ds(row0 + p * r_grp, r_grp), :], bufs[p],
            sems[p]).start()

    lane = lax.iota(jnp.int32, _L)
    svec = jnp.zeros((_L,), jnp.float32)
    cur16 = jnp.zeros((_L,), jnp.float32)
    for g in range(ngrp):
        cur = bufs[g % _NBUF]
        pltpu.make_async_copy(
            a_hbm.at[pl.ds(row0 + g * r_grp, r_grp), :], cur,
            sems[g % _NBUF]).wait()
        nxt = g + _NBUF - 1
        if nxt < ngrp:
            pltpu.make_async_copy(
                a_hbm.at[pl.ds(row0 + nxt * r_grp, r_grp), :],
                bufs[nxt % _NBUF], sems[nxt % _NBUF]).start()

        zero = jnp.zeros((_L,), jnp.float32)
        accs_init = (zero,) * r_grp
        unroll = 4

        def jbody(jj, accs, cur=cur):
            for u in range(unroll):
                col = (jj * unroll + u) * _L
                xc = xv[pl.ds(col, _L)]
                accs = tuple(
                    accs[r] + cur[r, pl.ds(col, _L)] * xc
                    for r in range(r_grp))
            return accs

        accs = lax.fori_loop(0, nj // unroll, jbody, accs_init)
        for r in range(r_grp):
            row = g * r_grp + r
            v = accs[r]
            for k in (8, 4, 2, 1):
                v = v + _permute(v, lane ^ k)
            cur16 = jnp.where(lane == (row % _L), v, cur16)
            if (row + 1) % _L == 0:
                base = row + 1 - _L
                t = yv[pl.ds(base, _L)] * (cur16 - bv[pl.ds(base, _L)])
                t = t * iyv[pl.ds(base, _L)]
                svec = svec + jnp.abs(t)
                cur16 = jnp.zeros((_L,), jnp.float32)

    sv[...] = svec
    pltpu.sync_copy(sv, out_hbm.at[wid])


def _sc_axb(A, svec, b):
    m, n = A.shape
    rows_per_w = m // _NW
    r_grp = 8
    mesh = plsc.VectorSubcoreMesh(core_axis_name="c", subcore_axis_name="s")
    f = pl.kernel(
        functools.partial(_sc_axb_body, rows_per_w=rows_per_w, r_grp=r_grp),
        mesh=mesh,
        out_type=jax.ShapeDtypeStruct((_NW, _L), jnp.float32),
        scratch_types=[
            pltpu.VMEM((n,), jnp.float32),
            pltpu.VMEM((rows_per_w,), jnp.float32),
            pltpu.VMEM((rows_per_w,), jnp.float32),
            pltpu.VMEM((rows_per_w,), jnp.float32),
            pltpu.VMEM((r_grp, n), jnp.float32),
            pltpu.VMEM((r_grp, n), jnp.float32),
            pltpu.VMEM((r_grp, n), jnp.float32),
            pltpu.VMEM((_L,), jnp.float32),
            pltpu.SemaphoreType.DMA,
            pltpu.SemaphoreType.DMA,
            pltpu.SemaphoreType.DMA,
        ],
    )
    return f(A, svec, b)


_SPLIT = 2


def _tc_body(at0_ref, at1_ref, q0_ref, q1_ref, w_ref, out_ref, acc_ref,
             *, blk, nsteps):
    i = pl.program_id(0)

    xv = w_ref[:, 0:1]
    yv = w_ref[:, 1:2]

    s = jnp.float32(0.0)
    for k, (at_ref, q_ref) in enumerate(((at0_ref, q0_ref),
                                         (at1_ref, q1_ref))):
        aty = jnp.dot(at_ref[...], yv, preferred_element_type=jnp.float32)
        qx = jnp.dot(q_ref[...], xv, preferred_element_type=jnp.float32)

        wb = w_ref[pl.ds((_SPLIT * i + k) * blk, blk), :]
        pg = wb[:, 2:3] - aty + qx

        x_blk = wb[:, 0:1]
        lb = jnp.maximum(pg, 0.0) * wb[:, 3:4]
        s = s + jnp.sum(jnp.abs((x_blk - wb[:, 5:6]) * lb))
        ub = jnp.maximum(-pg, 0.0) * wb[:, 4:5]
        s = s + jnp.sum(jnp.abs((wb[:, 6:7] - x_blk) * ub))

    @pl.when(i == 0)
    def _():
        acc_ref[0] = 0.0

    acc_ref[0] += s

    @pl.when(i == nsteps - 1)
    def _():
        out_ref[...] = jnp.full((1, 1), acc_ref[0], dtype=jnp.float32)


def _tc_rc(Q, AT, W):
    n = Q.shape[0]
    m = AT.shape[1]
    blk = 256
    nsteps = n // (blk * _SPLIT)

    spec = lambda cols, k: pl.BlockSpec(
        (blk, cols), lambda i, k=k: (_SPLIT * i + k, 0))

    return pl.pallas_call(
        functools.partial(_tc_body, blk=blk, nsteps=nsteps),
        grid=(nsteps,),
        in_specs=[
            spec(m, 0),
            spec(m, 1),
            spec(n, 0),
            spec(n, 1),
            pl.BlockSpec((n, 7), lambda i: (0, 0)),
        ],
        out_specs=pl.BlockSpec((1, 1), lambda i: (0, 0)),
        out_shape=jax.ShapeDtypeStruct((1, 1), jnp.float32),
        scratch_shapes=[pltpu.SMEM((1,), jnp.float32)],
    )(AT, AT, Q, Q, W)


def kernel(Q, A, AT, b, c, x, y, Iy, il, iu, l, u):
    svec = jnp.concatenate([x[:, 0], y[:, 0], Iy[:, 0]])
    W = jnp.concatenate([x, y, c[:, None], il, iu, l, u], axis=1)
    sc_out = _sc_axb(A, svec, b)
    tc_out = _tc_rc(Q, AT, W)
    return (jnp.sum(sc_out) + tc_out[0, 0]) * (1.0 / _ETA_OPT)
